# baseline (speedup 1.0000x reference)
import functools

import jax
import jax.numpy as jnp
from jax.experimental import pallas as pl
from jax.experimental.pallas import tpu as pltpu


def _attn_kernel(x_ref, watt_ref, wenc_ref, benc_ref, out_ref, emb_ref, *, bt):
    watt = watt_ref[...]
    wenc = wenc_ref[...]
    for i in range(bt):
        x = x_ref[i]
        att = jax.lax.dot_general(x, watt, (((1,), (1,)), ((), ())))
        att = att - jnp.max(att, axis=0, keepdims=True)
        e = jnp.exp(att)
        sm = e / jnp.sum(e, axis=0, keepdims=True)
        out = jax.lax.dot_general(sm, x, (((0,), (0,)), ((), ())))
        out_ref[:, i, :] = out
        emb = jax.lax.dot_general(out, wenc, (((1,), (1,)), ((), ())))
        emb_ref[i] = emb + benc_ref[...]


def _clf_kernel(out_ref, wclf_ref, bclf_ref, c_ref, *, a_heads):
    wclf = wclf_ref[...]
    out512 = jnp.concatenate([out_ref[a] for a in range(a_heads)], axis=0)
    logits = jax.lax.dot_general(out512, wclf, (((1,), (1,)), ((), ())))
    b = out_ref.shape[1]
    acc = logits[0:b]
    for a in range(1, a_heads):
        acc = jnp.maximum(acc, logits[a * b:(a + 1) * b])
    c_ref[...] = acc + bclf_ref[...]


def _topk_kernel(c_ref, idx_ref, sc3_ref, cand3_ref, *, topk, l_sz):
    vals = c_ref[...]
    b, g = vals.shape
    iota_g = jax.lax.broadcasted_iota(jnp.int32, (b, g), 1)
    idx_cols, sc_cols = [], []
    for _ in range(topk):
        m = jnp.max(vals, axis=1, keepdims=True)
        idx = jnp.min(jnp.where(vals == m, iota_g, g), axis=1, keepdims=True)
        idx_cols.append(idx)
        sc_cols.append(jax.nn.sigmoid(m))
        vals = jnp.where(iota_g == idx, -jnp.inf, vals)
    idx_all = jnp.concatenate(idx_cols, axis=1)
    idx_ref[...] = idx_all
    sc_all = jnp.concatenate(sc_cols, axis=1)
    sc3_ref[...] = jnp.broadcast_to(sc_all[:, :, None], (b, topk, l_sz))
    iota_l = jax.lax.broadcasted_iota(jnp.int32, (b, topk, l_sz), 2)
    cand3_ref[...] = idx_all[:, :, None] * l_sz + iota_l


def _score_kernel(idx_ref, emb_ref, *rest, topk, l_sz, bd):
    blk_refs, can_ref = rest[:bd * topk], rest[bd * topk]
    for i in range(bd):
        e = emb_ref[i]
        for k in range(topk):
            blk = blk_refs[i * topk + k][...]
            l2 = jax.lax.dot_general(blk, e, (((1,), (1,)), ((), ())))
            can_ref[i, 0, k * l_sz:(k + 1) * l_sz] = jnp.max(l2, axis=1)


def _attn_call(inputs, W_att, W_enc, b_enc):
    B, S, H = inputs.shape
    A = W_att.shape[0]
    D2 = W_enc.shape[0]
    BT = 8
    return pl.pallas_call(
        functools.partial(_attn_kernel, bt=BT),
        grid=(B // BT,),
        in_specs=[
            pl.BlockSpec((BT, S, H), lambda b: (b, 0, 0)),
            pl.BlockSpec((A, H), lambda b: (0, 0)),
            pl.BlockSpec((D2, H), lambda b: (0, 0)),
            pl.BlockSpec((1, D2), lambda b: (0, 0)),
        ],
        out_specs=[
            pl.BlockSpec((A, BT, H), lambda b: (0, b, 0)),
            pl.BlockSpec((BT, A, D2), lambda b: (b, 0, 0)),
        ],
        out_shape=[
            jax.ShapeDtypeStruct((A, B, H), jnp.float32),
            jax.ShapeDtypeStruct((B, A, D2), jnp.float32),
        ],
    )(inputs, W_att, W_enc, b_enc.reshape(1, D2))


def _clf_call(out_abh, W_clf, b_clf):
    A, B, H = out_abh.shape
    G = W_clf.shape[0]
    return pl.pallas_call(
        functools.partial(_clf_kernel, a_heads=A),
        grid=(1,),
        in_specs=[
            pl.BlockSpec((A, B, H), lambda g: (0, 0, 0)),
            pl.BlockSpec((G, H), lambda g: (0, 0)),
            pl.BlockSpec((1, G), lambda g: (0, 0)),
        ],
        out_specs=pl.BlockSpec((B, G), lambda g: (0, 0)),
        out_shape=jax.ShapeDtypeStruct((B, G), jnp.float32),
    )(out_abh, W_clf, b_clf.reshape(1, G))


def _topk_call(c_out, TOPK, L):
    B, G = c_out.shape
    return pl.pallas_call(
        functools.partial(_topk_kernel, topk=TOPK, l_sz=L),
        grid=(1,),
        in_specs=[pl.BlockSpec((B, G), lambda i: (0, 0))],
        out_specs=[
            pl.BlockSpec((B, TOPK), lambda i: (0, 0)),
            pl.BlockSpec((B, TOPK, L), lambda i: (0, 0, 0)),
            pl.BlockSpec((B, TOPK, L), lambda i: (0, 0, 0)),
        ],
        out_shape=[
            jax.ShapeDtypeStruct((B, TOPK), jnp.int32),
            jax.ShapeDtypeStruct((B, TOPK, L), jnp.float32),
            jax.ShapeDtypeStruct((B, TOPK, L), jnp.int32),
        ],
    )(c_out)


def _score_call(idx, emb, embed_table, TOPK, L):
    B, A, D2 = emb.shape
    C = TOPK * L
    BD = 4
    blk_specs = [
        pl.BlockSpec((L, D2),
                     functools.partial(
                         lambda b, ir, ii, kk: (ir[b * BD + ii, kk], 0),
                         ii=i, kk=k))
        for i in range(BD) for k in range(TOPK)
    ]
    return pl.pallas_call(
        functools.partial(_score_kernel, topk=TOPK, l_sz=L, bd=BD),
        grid_spec=pltpu.PrefetchScalarGridSpec(
            num_scalar_prefetch=1,
            grid=(B // BD,),
            in_specs=[pl.BlockSpec((BD, A, D2), lambda b, ir: (b, 0, 0))] + blk_specs,
            out_specs=pl.BlockSpec((BD, 1, C), lambda b, ir: (b, 0, 0)),
        ),
        out_shape=jax.ShapeDtypeStruct((B, 1, C), jnp.float32),
    )(idx, emb, *([embed_table] * (BD * TOPK)))


def kernel(inputs, labels, group_labels, candidates, W_att, b_att, W_clf,
           b_clf, W_enc, b_enc, embed_table, group_y):
    B, S, H = inputs.shape
    L = group_y.shape[1]
    TOPK = 10
    C = TOPK * L

    out_abh, emb = _attn_call(inputs, W_att, W_enc, b_enc)
    c_out = _clf_call(out_abh, W_clf, b_clf)
    idx, sc3, cand3 = _topk_call(c_out, TOPK, L)
    can3 = _score_call(idx, emb, embed_table, TOPK, L)

    return (c_out, can3.reshape(B, C), cand3.reshape(B, C).astype(jnp.int32),
            sc3.reshape(B, C))


# speedup vs baseline: 1.5568x; 1.5568x over previous
import functools

import jax
import jax.numpy as jnp
from jax import lax
from jax.experimental import pallas as pl
from jax.experimental.pallas import tpu as pltpu
from jax.experimental.pallas import tpu_sc as plsc


def _attn_kernel(x_ref, watt_ref, wenc_ref, benc_ref, out_ref, emb_ref, *, bt):
    watt = watt_ref[...]
    wenc = wenc_ref[...]
    for i in range(bt):
        x = x_ref[i]
        att = jax.lax.dot_general(x, watt, (((1,), (1,)), ((), ())))
        att = att - jnp.max(att, axis=0, keepdims=True)
        e = jnp.exp(att)
        sm = e / jnp.sum(e, axis=0, keepdims=True)
        out = jax.lax.dot_general(sm, x, (((0,), (0,)), ((), ())))
        out_ref[:, i, :] = out
        emb = jax.lax.dot_general(out, wenc, (((1,), (1,)), ((), ())))
        emb_ref[i] = emb + benc_ref[...]


def _clf_kernel(out_ref, wclf_ref, bclf_ref, c_ref, *, a_heads):
    wclf = wclf_ref[...]
    out512 = jnp.concatenate([out_ref[a] for a in range(a_heads)], axis=0)
    logits = jax.lax.dot_general(out512, wclf, (((1,), (1,)), ((), ())))
    b = out_ref.shape[1]
    acc = logits[0:b]
    for a in range(1, a_heads):
        acc = jnp.maximum(acc, logits[a * b:(a + 1) * b])
    c_ref[...] = acc + bclf_ref[...]


def _topk_kernel(c_ref, idx_ref, sc3_ref, cand3_ref, *, topk, l_sz):
    vals = c_ref[...]
    b, g = vals.shape
    iota_g = jax.lax.broadcasted_iota(jnp.int32, (b, g), 1)
    idx_cols, sc_cols = [], []
    for _ in range(topk):
        m = jnp.max(vals, axis=1, keepdims=True)
        idx = jnp.min(jnp.where(vals == m, iota_g, g), axis=1, keepdims=True)
        idx_cols.append(idx)
        sc_cols.append(jax.nn.sigmoid(m))
        vals = jnp.where(iota_g == idx, -jnp.inf, vals)
    idx_all = jnp.concatenate(idx_cols, axis=1)
    idx_ref[...] = idx_all
    sc_all = jnp.concatenate(sc_cols, axis=1)
    sc3_ref[...] = jnp.broadcast_to(sc_all[:, :, None], (b, topk, l_sz))
    iota_l = jax.lax.broadcasted_iota(jnp.int32, (b, topk, l_sz), 2)
    cand3_ref[...] = idx_all[:, :, None] * l_sz + iota_l


def _score_kernel(idx_ref, emb_ref, *rest, topk, l_sz, bd):
    blk_refs, can_ref = rest[:bd * topk], rest[bd * topk]
    for i in range(bd):
        e = emb_ref[i]
        for k in range(topk):
            blk = blk_refs[i * topk + k][0]
            l2 = jax.lax.dot_general(blk, e, (((1,), (1,)), ((), ())))
            can_ref[i, 0, k * l_sz:(k + 1) * l_sz] = jnp.max(l2, axis=1)


def _attn_call(inputs, W_att, W_enc, b_enc):
    B, S, H = inputs.shape
    A = W_att.shape[0]
    D2 = W_enc.shape[0]
    BT = 8
    return pl.pallas_call(
        functools.partial(_attn_kernel, bt=BT),
        grid=(B // BT,),
        in_specs=[
            pl.BlockSpec((BT, S, H), lambda b: (b, 0, 0)),
            pl.BlockSpec((A, H), lambda b: (0, 0)),
            pl.BlockSpec((D2, H), lambda b: (0, 0)),
            pl.BlockSpec((1, D2), lambda b: (0, 0)),
        ],
        out_specs=[
            pl.BlockSpec((A, BT, H), lambda b: (0, b, 0)),
            pl.BlockSpec((BT, A, D2), lambda b: (b, 0, 0)),
        ],
        out_shape=[
            jax.ShapeDtypeStruct((A, B, H), jnp.float32),
            jax.ShapeDtypeStruct((B, A, D2), jnp.float32),
        ],
    )(inputs, W_att, W_enc, b_enc.reshape(1, D2))


def _clf_call(out_abh, W_clf, b_clf):
    A, B, H = out_abh.shape
    G = W_clf.shape[0]
    return pl.pallas_call(
        functools.partial(_clf_kernel, a_heads=A),
        grid=(1,),
        in_specs=[
            pl.BlockSpec((A, B, H), lambda g: (0, 0, 0)),
            pl.BlockSpec((G, H), lambda g: (0, 0)),
            pl.BlockSpec((1, G), lambda g: (0, 0)),
        ],
        out_specs=pl.BlockSpec((B, G), lambda g: (0, 0)),
        out_shape=jax.ShapeDtypeStruct((B, G), jnp.float32),
    )(out_abh, W_clf, b_clf.reshape(1, G))


def _topk_call(c_out, TOPK, L):
    B, G = c_out.shape
    return pl.pallas_call(
        functools.partial(_topk_kernel, topk=TOPK, l_sz=L),
        grid=(1,),
        in_specs=[pl.BlockSpec((B, G), lambda i: (0, 0))],
        out_specs=[
            pl.BlockSpec((B, TOPK), lambda i: (0, 0)),
            pl.BlockSpec((B, TOPK, L), lambda i: (0, 0, 0)),
            pl.BlockSpec((B, TOPK, L), lambda i: (0, 0, 0)),
        ],
        out_shape=[
            jax.ShapeDtypeStruct((B, TOPK), jnp.int32),
            jax.ShapeDtypeStruct((B, TOPK, L), jnp.float32),
            jax.ShapeDtypeStruct((B, TOPK, L), jnp.int32),
        ],
    )(c_out)


def _score_call(idx, emb, embed_table, TOPK, L):
    B, A, D2 = emb.shape
    C = TOPK * L
    BD = 4
    blk_specs = [
        pl.BlockSpec((1, L, D2),
                     functools.partial(
                         lambda b, ir, ii, kk: (ir[b * BD + ii, kk], 0, 0),
                         ii=i, kk=k))
        for i in range(BD) for k in range(TOPK)
    ]
    return pl.pallas_call(
        functools.partial(_score_kernel, topk=TOPK, l_sz=L, bd=BD),
        grid_spec=pltpu.PrefetchScalarGridSpec(
            num_scalar_prefetch=1,
            grid=(B // BD,),
            in_specs=[pl.BlockSpec((BD, A, D2), lambda b, ir: (b, 0, 0))] + blk_specs,
            out_specs=pl.BlockSpec((BD, 1, C), lambda b, ir: (b, 0, 0)),
        ),
        out_shape=jax.ShapeDtypeStruct((B, 1, C), jnp.float32),
    )(idx, emb, *([embed_table.reshape(-1, L, D2)] * (BD * TOPK)))


def _gather_sc(embed_table, cand):
    B, C = cand.shape
    D2 = embed_table.shape[1]
    info = plsc.get_sparse_core_info()
    NC, NS = info.num_cores, info.num_subcores
    NW = NC * NS
    CHUNK = 256
    CH = C // CHUNK
    NCHUNKS = B * CH

    @functools.partial(
        pl.kernel,
        mesh=plsc.VectorSubcoreMesh(core_axis_name="c", subcore_axis_name="s"),
        out_type=jax.ShapeDtypeStruct((B, C, D2), jnp.float32),
        scratch_types=[
            pltpu.VMEM((CHUNK,), jnp.int32),
            pltpu.VMEM((CHUNK, D2), jnp.float32),
            pltpu.SemaphoreType.DMA,
        ],
    )
    def k(table_hbm, cand_hbm, out_hbm, idx_v, rows_v, sem):
        wid = lax.axis_index("s") * NC + lax.axis_index("c")

        def body(i, _):
            j = wid + i * NW
            b = j // CH
            ch = j % CH
            pltpu.sync_copy(cand_hbm.at[b, pl.ds(ch * CHUNK, CHUNK)], idx_v)
            pltpu.async_copy(table_hbm.at[idx_v], rows_v, sem).wait()
            pltpu.sync_copy(rows_v, out_hbm.at[b, pl.ds(ch * CHUNK, CHUNK)])
            return _

        lax.fori_loop(0, NCHUNKS // NW, body, None)

    return k(embed_table, cand)


def _score2_kernel(gath_ref, emb_ref, can_ref):
    e = emb_ref[0]
    l2 = jax.lax.dot_general(gath_ref[0], e, (((1,), (1,)), ((), ())))
    can_ref[0, 0] = jnp.max(l2, axis=1)


def _score2_call(gath, emb):
    B, C, D2 = gath.shape
    A = emb.shape[1]
    return pl.pallas_call(
        _score2_kernel,
        grid=(B,),
        in_specs=[
            pl.BlockSpec((1, C, D2), lambda b: (b, 0, 0)),
            pl.BlockSpec((1, A, D2), lambda b: (b, 0, 0)),
        ],
        out_specs=pl.BlockSpec((1, 1, C), lambda b: (b, 0, 0)),
        out_shape=jax.ShapeDtypeStruct((B, 1, C), jnp.float32),
    )(gath, emb)


def kernel(inputs, labels, group_labels, candidates, W_att, b_att, W_clf,
           b_clf, W_enc, b_enc, embed_table, group_y):
    B, S, H = inputs.shape
    L = group_y.shape[1]
    TOPK = 10
    C = TOPK * L

    out_abh, emb = _attn_call(inputs, W_att, W_enc, b_enc)
    c_out = _clf_call(out_abh, W_clf, b_clf)
    idx, sc3, cand3 = _topk_call(c_out, TOPK, L)
    cand = cand3.reshape(B, C)
    can3 = _score_call(idx, emb, embed_table, TOPK, L)

    return (c_out, can3.reshape(B, C), cand, sc3.reshape(B, C))
